# baseline (speedup 1.0000x reference)
import functools

import jax
import jax.numpy as jnp
from jax import lax
from jax.experimental import pallas as pl
from jax.experimental.pallas import tpu as pltpu
from jax.experimental.pallas import tpu_sc as plsc

_BINS = 100
_HMIN = 60.0
_HMAX = 120.0

_NC = 2
_NS = 16
_NW = _NC * _NS
_L = 16

_CE = 2048
_DT_OFF = 128

_C2 = 1.0 / 24.0
_C3 = -1.0 / 720.0
_C4 = 1.0 / 40320.0
_C5 = -1.0 / 3628800.0


def _cos_poly(x):
    u = x * x
    return ((((_C5 * u + _C4) * u + _C3) * u + _C2) * u - 0.5) * u + 1.0


def _sqrt_sc(x):
    i = lax.bitcast_convert_type(x, jnp.int32)
    i = 0x5F3759DF - lax.shift_right_logical(i, 1)
    y = lax.bitcast_convert_type(i, jnp.float32)
    xh = 0.5 * x
    y = y * (1.5 - xh * y * y)
    y = y * (1.5 - xh * y * y)
    y = y * (1.5 - xh * y * y)
    return x * y


def _bin_index(x):
    t = (x - _HMIN) * _BINS / (_HMAX - _HMIN)
    i0 = jnp.minimum(jnp.maximum(t.astype(jnp.int32), 0), _BINS - 1)
    valid = (x >= _HMIN) & (x <= _HMAX)
    return jnp.where(valid, i0, _BINS)


def _make_sc_body(n):
    ev = n // _NW
    nch = ev // _CE
    tail = ev - nch * _CE
    tvec = tail // 16
    trem = tail - tvec * 16
    max_extra = max(((w * ev + nch * _CE) * 9) % 8 for w in range(_NW))
    dt_full_sz = _CE * 9 + 8
    dt_tail_sz = tail * 9 + max_extra if tail else 8

    def body(corr_hbm, mc_hbm, dt_hbm, out_hbm, bc, bm, bd, hist):
        wid = lax.axis_index("s") * _NC + lax.axis_index("c")
        wbase = wid * ev

        lane = lax.broadcasted_iota(jnp.int32, (_L,), 0)
        pat2 = 2 * lane
        pat8 = 8 * lane
        pat9 = 9 * lane
        ones = jnp.full((_L,), 1.0, jnp.float32)
        zeros = jnp.zeros((_L,), jnp.float32)
        mc_base = lane * 256
        dt_base = lane * 256 + _DT_OFF

        for k in range(_L * 256 // _L):
            hist[pl.ds(k * _L, _L)] = zeros

        def do_vec(j, extra, mask):
            g = lambda ref, pat, s: plsc.load_gather(ref, [pat + s], mask=mask)
            c0 = g(bc, pat2, 32 * j)
            c1 = g(bc, pat2, 32 * j + 1)
            m0 = g(bm, pat8, 128 * j)
            m1 = g(bm, pat8, 128 * j + 1)
            f1 = g(bm, pat8, 128 * j + 4)
            f2 = g(bm, pat8, 128 * j + 5)
            e1 = g(bm, pat8, 128 * j + 6)
            e2 = g(bm, pat8, 128 * j + 7)
            xd = g(bd, pat9, 144 * j + 8 + extra)

            q = (c0 * c1) * (m0 * m1)
            de = e1 - e2
            ep = jnp.exp(de)
            em = jnp.exp(-de)
            ch = 0.5 * (ep + em)
            co = _cos_poly(f1 - f2)
            mz2 = 2.0 * q * (ch - co)
            mz = _sqrt_sc(jnp.maximum(mz2, 1e-30))

            imc = _bin_index(mz)
            idt = _bin_index(xd)
            plsc.addupdate_scatter(hist, [mc_base + imc], ones, mask=mask)
            plsc.addupdate_scatter(hist, [dt_base + idt], ones, mask=mask)

        def do_chunk(ebase, nvec, dt_sz):
            off_c = pl.multiple_of(ebase * 2, 8)
            off_m = pl.multiple_of(ebase * 8, 8)
            pltpu.sync_copy(corr_hbm.at[pl.ds(off_c, _CE * 2)], bc)
            pltpu.sync_copy(mc_hbm.at[pl.ds(off_m, _CE * 8)], bm)
            dstart = ebase * 9
            extra = lax.bitwise_and(dstart, 7)
            off_d = pl.multiple_of(dstart - extra, 8)
            pltpu.sync_copy(dt_hbm.at[pl.ds(off_d, dt_sz)],
                            bd.at[pl.ds(0, dt_sz)])

            def vec_body(j, carry):
                for u in range(4):
                    do_vec(4 * j + u, extra, None)
                return carry

            lax.fori_loop(0, nvec // 4, vec_body, 0)
            for j in range(nvec - nvec % 4, nvec):
                do_vec(j, extra, None)
            return extra

        def chunk_body(k, carry):
            do_chunk(wbase + k * _CE, _CE // 16, dt_full_sz)
            return carry

        lax.fori_loop(0, nch, chunk_body, 0)

        if tail:
            extra = do_chunk(wbase + nch * _CE, tvec, dt_tail_sz)
            if trem:
                do_vec(tvec, extra, lane < trem)

        pltpu.sync_copy(hist, out_hbm.at[wid])

    return body


def _kl_body(h_ref, out_ref):
    s = jnp.sum(h_ref[...], axis=0, keepdims=True)
    hm = s[:, 0:_BINS]
    hd = s[:, _DT_OFF:_DT_OFF + _BINS]
    pw = jnp.where(hd > 0.0, hd * (jnp.log(jnp.where(hd > 0.0, hd, 1.0)) - hm), 0.0)
    out_ref[...] = (jnp.sum(pw) / float(_BINS)).reshape(1, 1)


def kernel(corr, mc, dt):
    n = corr.shape[0]

    sc_hist = functools.partial(
        pl.kernel,
        mesh=plsc.VectorSubcoreMesh(core_axis_name="c", subcore_axis_name="s"),
        out_type=jax.ShapeDtypeStruct((_NW, _L * 256), jnp.float32),
        scratch_types=[
            pltpu.VMEM((_CE * 2,), jnp.float32),
            pltpu.VMEM((_CE * 8,), jnp.float32),
            pltpu.VMEM((_CE * 9 + 8,), jnp.float32),
            pltpu.VMEM((_L * 256,), jnp.float32),
        ],
        compiler_params=pltpu.CompilerParams(needs_layout_passes=False),
    )(_make_sc_body(n))
    hists = sc_hist(corr.reshape(-1), mc.reshape(-1), dt.reshape(-1))

    out = pl.pallas_call(
        _kl_body,
        in_specs=[pl.BlockSpec((_NW * _L, 256), lambda: (0, 0))],
        out_specs=pl.BlockSpec((1, 1), lambda: (0, 0)),
        out_shape=jax.ShapeDtypeStruct((1, 1), jnp.float32),
    )(hists.reshape(_NW * _L, 256))
    return out[0, 0]
